# Initial kernel scaffold; baseline (speedup 1.0000x reference)
import jax
import jax.numpy as jnp
from jax.experimental import pallas as pl


def kernel(cls_global, labels, centers):
    raise NotImplementedError("write your pallas kernel here")


# speedup vs baseline: 3.1221x; 3.1221x over previous
import jax
import jax.numpy as jnp
from jax import lax
from jax.experimental import pallas as pl
from jax.experimental.pallas import tpu as pltpu

_B = 16384
_D = 128
_NC = 8
_GAMMA2 = 0.25
_CENTER_MARGIN = 1.0
_LAMBDA_CENTER = 0.001
_EPS = 1e-06

_CHUNK = 2048
_NCHUNK = _B // _CHUNK
_INF_BITS = 0x7F800000


def _body(labels_ref, x_ref, centers_ref, out_ref, d2_ref):
    i = pl.program_id(0)
    x = x_ref[...]
    c = centers_ref[...]
    xc = lax.dot_general(x, c, (((1,), (1,)), ((), ())),
                         preferred_element_type=jnp.float32)
    xn = jnp.sum(x * x, axis=1, keepdims=True)
    cn = jnp.sum(c * c, axis=1)[None, :]
    d2all = xn - 2.0 * xc + cn
    d2 = jnp.maximum(jnp.min(d2all, axis=1), 0.0)
    d2_ref[i, :] = d2

    @pl.when(i == _NCHUNK - 1)
    def _select():
        d2a = d2_ref[...]
        lab = labels_ref[...]
        real = lab == 0
        forged = lab == 1
        bits = lax.bitcast_convert_type(d2a, jnp.int32)

        num_real_f = jnp.sum(jnp.where(real, 1.0, 0.0))
        num_forged_f = jnp.sum(jnp.where(forged, 1.0, 0.0))
        num_real = num_real_f.astype(jnp.int32)
        num_forged = num_forged_f.astype(jnp.int32)
        k_real = jnp.maximum(1, (7 * num_real + 9) // 10)
        k_forged = jnp.maximum(1, (7 * num_forged + 9) // 10)
        k_real_f = k_real.astype(jnp.float32)
        k_forged_f = k_forged.astype(jnp.float32)

        def it(_, carry):
            lo_r, hi_r, lo_f, hi_f = carry
            mid_r = lo_r + (hi_r - lo_r) // 2
            mid_f = lo_f + (hi_f - lo_f) // 2
            cnt_r = jnp.sum(jnp.where(real & (bits >= mid_r), 1.0, 0.0))
            cnt_f = jnp.sum(jnp.where(forged & (bits < mid_f), 1.0, 0.0))
            ge = cnt_r >= k_real_f
            lo_r = jnp.where(ge, mid_r, lo_r)
            hi_r = jnp.where(ge, hi_r, mid_r)
            lt = cnt_f < k_forged_f
            lo_f = jnp.where(lt, mid_f, lo_f)
            hi_f = jnp.where(lt, hi_f, mid_f)
            return lo_r, hi_r, lo_f, hi_f

        zero = jnp.int32(0)
        hi0 = jnp.int32(_INF_BITS)
        lo_r, _, lo_f, _ = lax.fori_loop(0, 31, it, (zero, hi0, zero, hi0))

        v_r = lax.bitcast_convert_type(lo_r, jnp.float32)
        gt = real & (bits > lo_r)
        sum_gt = jnp.sum(jnp.where(gt, d2a, 0.0))
        cnt_gt = jnp.sum(jnp.where(gt, 1.0, 0.0))
        top_sum = sum_gt + (k_real_f - cnt_gt) * v_r
        real_loss = top_sum / (2.0 * (k_real_f + _EPS))
        real_loss = jnp.where(num_real > 0, real_loss, 0.0)

        v_f = lax.bitcast_convert_type(lo_f, jnp.float32)
        ltm = forged & (bits < lo_f)
        sum_lt = jnp.sum(jnp.where(ltm, d2a, 0.0))
        cnt_lt = jnp.sum(jnp.where(ltm, 1.0, 0.0))
        bot_sum = sum_lt + (k_forged_f - cnt_lt) * v_f
        avg_forged = bot_sum / (2.0 * (k_forged_f + _EPS))
        forged_term = jnp.where(num_forged > 0,
                                jnp.minimum(avg_forged, _GAMMA2), 0.0)

        cc = lax.dot_general(c, c, (((1,), (1,)), ((), ())),
                             preferred_element_type=jnp.float32)
        cn2 = jnp.sum(c * c, axis=1)
        d2m = jnp.maximum(cn2[:, None] + cn2[None, :] - 2.0 * cc, 0.0)
        ii = lax.broadcasted_iota(jnp.int32, (_NC, _NC), 0)
        jj = lax.broadcasted_iota(jnp.int32, (_NC, _NC), 1)
        upper = jj > ii
        dist = jnp.sqrt(d2m + _EPS)
        hinge = jnp.maximum(_CENTER_MARGIN - dist, 0.0)
        num_pairs = _NC * (_NC - 1) // 2
        repulsion = _LAMBDA_CENTER * (
            jnp.sum(jnp.where(upper, hinge, 0.0)) / (num_pairs + _EPS))

        out_ref[0, 0] = real_loss - forged_term + repulsion


def kernel(cls_global, labels, centers):
    labels2d = labels.reshape(_NCHUNK, _CHUNK)
    out = pl.pallas_call(
        _body,
        grid=(_NCHUNK,),
        in_specs=[
            pl.BlockSpec((_NCHUNK, _CHUNK), lambda i: (0, 0)),
            pl.BlockSpec((_CHUNK, _D), lambda i: (i, 0)),
            pl.BlockSpec((_NC, _D), lambda i: (0, 0)),
        ],
        out_specs=pl.BlockSpec(memory_space=pltpu.SMEM),
        out_shape=jax.ShapeDtypeStruct((1, 1), jnp.float32),
        scratch_shapes=[pltpu.VMEM((_NCHUNK, _CHUNK), jnp.float32)],
        compiler_params=pltpu.CompilerParams(
            dimension_semantics=("arbitrary",),
        ),
    )(labels2d, cls_global, centers)
    return out[0, 0]


# speedup vs baseline: 4.9296x; 1.5790x over previous
import jax
import jax.numpy as jnp
from jax import lax
from jax.experimental import pallas as pl
from jax.experimental.pallas import tpu as pltpu

_B = 16384
_D = 128
_NC = 8
_GAMMA2 = 0.25
_CENTER_MARGIN = 1.0
_LAMBDA_CENTER = 0.001
_EPS = 1e-06

_CHUNK = 2048
_NCHUNK = _B // _CHUNK
_INF_BITS = 0x7F800000


def _body(labels_ref, x_ref, centers_ref, out_ref, d2_ref):
    i = pl.program_id(0)
    x = x_ref[...]
    c = centers_ref[...]
    cxT = lax.dot_general(c, x, (((1,), (1,)), ((), ())),
                          preferred_element_type=jnp.float32)
    ones = jnp.ones((1, _D), jnp.float32)
    xnT = lax.dot_general(ones, x * x, (((1,), (1,)), ((), ())),
                          preferred_element_type=jnp.float32)
    cn = jnp.sum(c * c, axis=1, keepdims=True)
    g = jnp.min(cn - 2.0 * cxT, axis=0, keepdims=True)
    d2_ref[pl.ds(i, 1), :] = g + xnT

    @pl.when(i == _NCHUNK - 1)
    def _select():
        d2a = jnp.maximum(d2_ref[...], 0.0)
        lab = labels_ref[...]
        real = lab == 0
        forged = lab == 1
        bits = lax.bitcast_convert_type(d2a, jnp.int32)
        rbits = jnp.where(real, bits, jnp.int32(-1))
        fbits = jnp.where(forged, bits, jnp.int32(0x7FFFFFFF))

        num_real_f = jnp.sum(jnp.where(real, 1.0, 0.0))
        num_real = num_real_f.astype(jnp.int32)
        num_forged = _B - num_real
        k_real = jnp.maximum(1, (7 * num_real + 9) // 10)
        k_forged = jnp.maximum(1, (7 * num_forged + 9) // 10)
        k_real_f = k_real.astype(jnp.float32)
        k_forged_f = k_forged.astype(jnp.float32)

        bmin = lax.bitcast_convert_type(jnp.min(d2a), jnp.int32)
        bmax = lax.bitcast_convert_type(jnp.max(d2a), jnp.int32) + 1

        def it(_, carry):
            lo_r, hi_r, lo_f, hi_f = carry
            mid_r = lo_r + (hi_r - lo_r) // 2
            mid_f = lo_f + (hi_f - lo_f) // 2
            cnt_r = jnp.sum(jnp.where(rbits >= mid_r, 1.0, 0.0))
            cnt_f = jnp.sum(jnp.where(fbits < mid_f, 1.0, 0.0))
            ge = cnt_r >= k_real_f
            lo_r = jnp.where(ge, mid_r, lo_r)
            hi_r = jnp.where(ge, hi_r, mid_r)
            lt = cnt_f < k_forged_f
            lo_f = jnp.where(lt, mid_f, lo_f)
            hi_f = jnp.where(lt, hi_f, mid_f)
            return lo_r, hi_r, lo_f, hi_f

        lo_r, _, lo_f, _ = lax.fori_loop(0, 20, it, (bmin, bmax, bmin, bmax))

        v_r = lax.bitcast_convert_type(lo_r, jnp.float32)
        gt = rbits > lo_r
        sum_gt = jnp.sum(jnp.where(gt, d2a, 0.0))
        cnt_gt = jnp.sum(jnp.where(gt, 1.0, 0.0))
        top_sum = sum_gt + (k_real_f - cnt_gt) * v_r
        real_loss = top_sum / (2.0 * (k_real_f + _EPS))
        real_loss = jnp.where(num_real > 0, real_loss, 0.0)

        v_f = lax.bitcast_convert_type(lo_f, jnp.float32)
        ltm = fbits < lo_f
        sum_lt = jnp.sum(jnp.where(ltm, d2a, 0.0))
        cnt_lt = jnp.sum(jnp.where(ltm, 1.0, 0.0))
        bot_sum = sum_lt + (k_forged_f - cnt_lt) * v_f
        avg_forged = bot_sum / (2.0 * (k_forged_f + _EPS))
        forged_term = jnp.where(num_forged > 0,
                                jnp.minimum(avg_forged, _GAMMA2), 0.0)

        cc = lax.dot_general(c, c, (((1,), (1,)), ((), ())),
                             preferred_element_type=jnp.float32)
        cn2 = jnp.sum(c * c, axis=1)
        d2m = jnp.maximum(cn2[:, None] + cn2[None, :] - 2.0 * cc, 0.0)
        ii = lax.broadcasted_iota(jnp.int32, (_NC, _NC), 0)
        jj = lax.broadcasted_iota(jnp.int32, (_NC, _NC), 1)
        upper = jj > ii
        dist = jnp.sqrt(d2m + _EPS)
        hinge = jnp.maximum(_CENTER_MARGIN - dist, 0.0)
        num_pairs = _NC * (_NC - 1) // 2
        repulsion = _LAMBDA_CENTER * (
            jnp.sum(jnp.where(upper, hinge, 0.0)) / (num_pairs + _EPS))

        out_ref[0, 0] = real_loss - forged_term + repulsion


def kernel(cls_global, labels, centers):
    labels2d = labels.reshape(_NCHUNK, _CHUNK)
    out = pl.pallas_call(
        _body,
        grid=(_NCHUNK,),
        in_specs=[
            pl.BlockSpec((_NCHUNK, _CHUNK), lambda i: (0, 0)),
            pl.BlockSpec((_CHUNK, _D), lambda i: (i, 0)),
            pl.BlockSpec((_NC, _D), lambda i: (0, 0)),
        ],
        out_specs=pl.BlockSpec(memory_space=pltpu.SMEM),
        out_shape=jax.ShapeDtypeStruct((1, 1), jnp.float32),
        scratch_shapes=[pltpu.VMEM((_NCHUNK, _CHUNK), jnp.float32)],
        compiler_params=pltpu.CompilerParams(
            dimension_semantics=("arbitrary",),
        ),
    )(labels2d, cls_global, centers)
    return out[0, 0]


# speedup vs baseline: 5.5409x; 1.1240x over previous
import jax
import jax.numpy as jnp
from jax import lax
from jax.experimental import pallas as pl
from jax.experimental.pallas import tpu as pltpu

_B = 16384
_D = 128
_NC = 8
_GAMMA2 = 0.25
_CENTER_MARGIN = 1.0
_LAMBDA_CENTER = 0.001
_EPS = 1e-06

_CHUNK = 4096
_NCHUNK = _B // _CHUNK
_INF_BITS = 0x7F800000


def _body(labels_ref, x_ref, centers_ref, out_ref, d2_ref):
    i = pl.program_id(0)
    x = x_ref[...]
    c = centers_ref[...]
    cxT = lax.dot_general(c, x, (((1,), (1,)), ((), ())),
                          preferred_element_type=jnp.float32)
    ones = jnp.ones((1, _D), jnp.float32)
    xnT = lax.dot_general(ones, x * x, (((1,), (1,)), ((), ())),
                          preferred_element_type=jnp.float32)
    cn = jnp.sum(c * c, axis=1, keepdims=True)
    g = jnp.min(cn - 2.0 * cxT, axis=0, keepdims=True)
    d2_ref[pl.ds(i, 1), :] = g + xnT

    @pl.when(i == _NCHUNK - 1)
    def _select():
        d2a = jnp.maximum(d2_ref[...], 0.0)
        lab = labels_ref[...]
        real = lab == 0
        forged = lab == 1
        bits = lax.bitcast_convert_type(d2a, jnp.int32)
        rbits = jnp.where(real, bits, jnp.int32(-1))
        fbits = jnp.where(forged, bits, jnp.int32(0x7FFFFFFF))

        num_real_f = jnp.sum(jnp.where(real, 1.0, 0.0))
        num_real = num_real_f.astype(jnp.int32)
        num_forged = _B - num_real
        k_real = jnp.maximum(1, (7 * num_real + 9) // 10)
        k_forged = jnp.maximum(1, (7 * num_forged + 9) // 10)
        k_real_f = k_real.astype(jnp.float32)
        k_forged_f = k_forged.astype(jnp.float32)

        bmin = lax.bitcast_convert_type(jnp.min(d2a), jnp.int32)
        bmax = lax.bitcast_convert_type(jnp.max(d2a), jnp.int32) + 1

        def it(_, carry):
            lo_r, hi_r, lo_f, hi_f = carry
            mid_r = lo_r + (hi_r - lo_r) // 2
            mid_f = lo_f + (hi_f - lo_f) // 2
            cnt_r = jnp.sum(jnp.where(rbits >= mid_r, 1.0, 0.0))
            cnt_f = jnp.sum(jnp.where(fbits < mid_f, 1.0, 0.0))
            ge = cnt_r >= k_real_f
            lo_r = jnp.where(ge, mid_r, lo_r)
            hi_r = jnp.where(ge, hi_r, mid_r)
            lt = cnt_f < k_forged_f
            lo_f = jnp.where(lt, mid_f, lo_f)
            hi_f = jnp.where(lt, hi_f, mid_f)
            return lo_r, hi_r, lo_f, hi_f

        lo_r, _, lo_f, _ = lax.fori_loop(0, 20, it, (bmin, bmax, bmin, bmax))

        v_r = lax.bitcast_convert_type(lo_r, jnp.float32)
        gt = rbits > lo_r
        sum_gt = jnp.sum(jnp.where(gt, d2a, 0.0))
        cnt_gt = jnp.sum(jnp.where(gt, 1.0, 0.0))
        top_sum = sum_gt + (k_real_f - cnt_gt) * v_r
        real_loss = top_sum / (2.0 * (k_real_f + _EPS))
        real_loss = jnp.where(num_real > 0, real_loss, 0.0)

        v_f = lax.bitcast_convert_type(lo_f, jnp.float32)
        ltm = fbits < lo_f
        sum_lt = jnp.sum(jnp.where(ltm, d2a, 0.0))
        cnt_lt = jnp.sum(jnp.where(ltm, 1.0, 0.0))
        bot_sum = sum_lt + (k_forged_f - cnt_lt) * v_f
        avg_forged = bot_sum / (2.0 * (k_forged_f + _EPS))
        forged_term = jnp.where(num_forged > 0,
                                jnp.minimum(avg_forged, _GAMMA2), 0.0)

        cc = lax.dot_general(c, c, (((1,), (1,)), ((), ())),
                             preferred_element_type=jnp.float32)
        cn2 = jnp.sum(c * c, axis=1)
        d2m = jnp.maximum(cn2[:, None] + cn2[None, :] - 2.0 * cc, 0.0)
        ii = lax.broadcasted_iota(jnp.int32, (_NC, _NC), 0)
        jj = lax.broadcasted_iota(jnp.int32, (_NC, _NC), 1)
        upper = jj > ii
        dist = jnp.sqrt(d2m + _EPS)
        hinge = jnp.maximum(_CENTER_MARGIN - dist, 0.0)
        num_pairs = _NC * (_NC - 1) // 2
        repulsion = _LAMBDA_CENTER * (
            jnp.sum(jnp.where(upper, hinge, 0.0)) / (num_pairs + _EPS))

        out_ref[0, 0] = real_loss - forged_term + repulsion


def kernel(cls_global, labels, centers):
    labels2d = labels.reshape(_NCHUNK, _CHUNK)
    out = pl.pallas_call(
        _body,
        grid=(_NCHUNK,),
        in_specs=[
            pl.BlockSpec((_NCHUNK, _CHUNK), lambda i: (0, 0)),
            pl.BlockSpec((_CHUNK, _D), lambda i: (i, 0)),
            pl.BlockSpec((_NC, _D), lambda i: (0, 0)),
        ],
        out_specs=pl.BlockSpec(memory_space=pltpu.SMEM),
        out_shape=jax.ShapeDtypeStruct((1, 1), jnp.float32),
        scratch_shapes=[pltpu.VMEM((_NCHUNK, _CHUNK), jnp.float32)],
        compiler_params=pltpu.CompilerParams(
            dimension_semantics=("arbitrary",),
        ),
    )(labels2d, cls_global, centers)
    return out[0, 0]
